# Initial kernel scaffold; baseline (speedup 1.0000x reference)
import jax
import jax.numpy as jnp
from jax.experimental import pallas as pl


def kernel(x, memory, mem_ts, mailbox, mail_ts, node_ts, edge_index, edge_feat, edge_ts, h_mem_hist, hist_ts, is_remote_mem, h_hist_next, hist_ts_next, is_remote_next, te_w, te_b, gru_w_ih, gru_w_hh, gru_b_ih, gru_b_hh, nfm_w, nfm_b, cte_w, cte_b, mc_w1, mc_b1, mc_w2, mc_b2, gc_w1, gc_b1, gc_w2, gc_b2, Wq, Wk, Wv, Wo, bo):
    raise NotImplementedError("write your pallas kernel here")


# speedup vs baseline: 17.3286x; 17.3286x over previous
import functools

import jax
import jax.numpy as jnp
from jax import lax
from jax.experimental import pallas as pl
from jax.experimental.pallas import tpu as pltpu
from jax.experimental.pallas import tpu_sc as plsc

N = 10000
E = 320000
H = 128
TF = 32
DE = 16
NH = 2
DH = 64

BN = 1000
NBN = N // BN
BE = 1600
NBE = E // BE
NW = 32
EPW = E // NW
CH = 80
NCHK = EPW // CH

DGW = 256
DEN = 2 * N
DENP = 20480
DSL = DENP // 16

_HI = jax.lax.Precision.HIGHEST


def _node_body(x_r, mem_r, mbx_r, hmh_r, sca_r, wiht_r, whht_r, bih_r, bhh_r,
               nfmt_r, nfmb_r, tew_r, teb_r, ctew_r, cteb_r, mw1_r, mb1_r,
               mw2_r, mb2_r, wkv_r, wqp_r, hin_o, gsv_o, dg_o):
    mem_ts = sca_r[:, 0:1]
    mail_ts = sca_r[:, 1:2]
    node_ts = sca_r[:, 2:3]
    hist_ts = sca_r[:, 3:4]
    rmask = sca_r[:, 4:5]
    tf = jnp.cos((mail_ts - mem_ts) * tew_r[0:1, :] + teb_r[0:1, :])
    xin = jnp.concatenate([mbx_r[...], tf], axis=1)
    gi = jnp.dot(xin, wiht_r[...], precision=_HI) + bih_r[0:1, :]
    gh = jnp.dot(mem_r[...], whht_r[...], precision=_HI) + bhh_r[0:1, :]
    r = jax.nn.sigmoid(gi[:, 0:H] + gh[:, 0:H])
    z = jax.nn.sigmoid(gi[:, H:2 * H] + gh[:, H:2 * H])
    nn = jnp.tanh(gi[:, 2 * H:3 * H] + r * gh[:, 2 * H:3 * H])
    out_mem = (1.0 - z) * nn + z * mem_r[...]
    h = out_mem + jnp.dot(x_r[...], nfmt_r[...], precision=_HI) + nfmb_r[0:1, :]
    dtm = jnp.maximum(node_ts - hist_ts, 0.0)
    tec = jnp.cos(dtm * ctew_r[0:1, :] + cteb_r[0:1, :])
    hcat = jnp.concatenate([hmh_r[...], tec], axis=1)
    hc = jnp.dot(jax.nn.relu(jnp.dot(hcat, mw1_r[...], precision=_HI) + mb1_r[0:1, :]),
                 mw2_r[...], precision=_HI) + mb2_r[0:1, :]
    hin = jnp.where(rmask > 0.5, hc, h)
    hin_o[...] = hin
    gsv_o[...] = jnp.dot(hin, wkv_r[...], precision=_HI)
    qp = jnp.dot(hin, wqp_r[...], precision=_HI)
    dg_o[...] = jnp.concatenate(
        [qp, node_ts, jnp.zeros((qp.shape[0], DGW - 225), jnp.float32)], axis=1)


def _node_pass(x, memory, mailbox, h_mem_hist, sca, wiht, whht, bih, bhh,
               nfmt, nfmb, tew, teb, ctew, cteb, mw1, mb1, mw2, mb2, wkv, wqp):
    full = lambda arr: pl.BlockSpec(arr.shape, lambda i: (0, 0))
    blk = lambda w: pl.BlockSpec((BN, w), lambda i: (i, 0))
    return pl.pallas_call(
        _node_body,
        grid=(NBN,),
        in_specs=[blk(H), blk(H), blk(2 * H), blk(H), blk(8),
                  full(wiht), full(whht), full(bih), full(bhh),
                  full(nfmt), full(nfmb), full(tew), full(teb),
                  full(ctew), full(cteb), full(mw1), full(mb1),
                  full(mw2), full(mb2), full(wkv), full(wqp)],
        out_specs=[blk(H), blk(2 * H), blk(DGW)],
        out_shape=[jax.ShapeDtypeStruct((N, H), jnp.float32),
                   jax.ShapeDtypeStruct((N, 2 * H), jnp.float32),
                   jax.ShapeDtypeStruct((N, DGW), jnp.float32)],
    )(x, memory, mailbox, h_mem_hist, sca, wiht, whht, bih, bhh,
      nfmt, nfmb, tew, teb, ctew, cteb, mw1, mb1, mw2, mb2, wkv, wqp)


def _sc_gather(gsv, dg, src3, dst3):
    mesh = plsc.VectorSubcoreMesh(core_axis_name="c", subcore_axis_name="s")

    @functools.partial(
        pl.kernel,
        out_type=(jax.ShapeDtypeStruct((E, 2 * H), jnp.float32),
                  jax.ShapeDtypeStruct((E, DGW), jnp.float32)),
        mesh=mesh,
        scratch_types=[pltpu.VMEM((NCHK, CH), jnp.int32),
                       pltpu.VMEM((NCHK, CH), jnp.int32),
                       pltpu.VMEM((CH, 2 * H), jnp.float32),
                       pltpu.VMEM((CH, DGW), jnp.float32),
                       pltpu.SemaphoreType.DMA,
                       pltpu.SemaphoreType.DMA],
    )
    def k(gsv_h, dg_h, src_h, dst_h, kvs_o, dge_o, src_v, dst_v, kv_b, dg_b,
          sem1, sem2):
        wid = lax.axis_index("c") * 16 + lax.axis_index("s")
        base = wid * EPW
        pltpu.sync_copy(src_h.at[wid], src_v)
        pltpu.sync_copy(dst_h.at[wid], dst_v)

        def body(i, carry):
            off = base + i * CH
            c1 = pltpu.async_copy(gsv_h.at[src_v.at[i]], kv_b, sem1)
            c2 = pltpu.async_copy(dg_h.at[dst_v.at[i]], dg_b, sem2)
            c1.wait()
            c2.wait()
            pltpu.sync_copy(kv_b, kvs_o.at[pl.ds(off, CH)])
            pltpu.sync_copy(dg_b, dge_o.at[pl.ds(off, CH)])
            return carry

        lax.fori_loop(0, NCHK, body, 0)

    return k(gsv, dg, src3, dst3)


def _edge_body(kvs_r, dge_r, ef_r, ets_r, tew_r, teb_r, wvet_r, rn_o, e_o):
    ts = dge_r[:, 224:225]
    dt = ts - ets_r[...]
    te = jnp.cos(dt * tew_r[0:1, :] + teb_r[0:1, :])
    et = jnp.concatenate([ef_r[...], te], axis=1)
    kv = kvs_r[...]
    dge = dge_r[...]
    s0 = (jnp.sum(dge[:, 0:DH] * kv[:, 0:DH], axis=1, keepdims=True)
          + jnp.sum(et * dge[:, 128:176], axis=1, keepdims=True))
    s1 = (jnp.sum(dge[:, DH:H] * kv[:, DH:H], axis=1, keepdims=True)
          + jnp.sum(et * dge[:, 176:224], axis=1, keepdims=True))
    e0 = jnp.exp(jnp.minimum(s0, 60.0))
    e1 = jnp.exp(jnp.minimum(s1, 60.0))
    v = kv[:, H:2 * H] + jnp.dot(et, wvet_r[...], precision=_HI)
    rn_o[...] = jnp.concatenate([e0 * v[:, 0:DH], e1 * v[:, DH:H]], axis=1)
    e_o[...] = jnp.concatenate([e0, e1], axis=1)


def _edge_pass(kvs, dge, ef, ets, tew, teb, wvet):
    full = lambda arr: pl.BlockSpec(arr.shape, lambda i: (0, 0))
    blk = lambda w: pl.BlockSpec((BE, w), lambda i: (i, 0))
    return pl.pallas_call(
        _edge_body,
        grid=(NBE,),
        in_specs=[blk(2 * H), blk(DGW), blk(DE), blk(1),
                  full(tew), full(teb), full(wvet)],
        out_specs=[blk(H), blk(2)],
        out_shape=[jax.ShapeDtypeStruct((E, H), jnp.float32),
                   jax.ShapeDtypeStruct((E, 2), jnp.float32)],
    )(kvs, dge, ef, ets, tew, teb, wvet)


def _sc_scatter(dst3, rnum, e0f, e1f, zn, zd):
    mesh = plsc.VectorSubcoreMesh(core_axis_name="c", subcore_axis_name="s")

    @functools.partial(
        pl.kernel,
        out_type=(jax.ShapeDtypeStruct((2, N, H), jnp.float32),
                  jax.ShapeDtypeStruct((2 * DENP,), jnp.float32)),
        mesh=mesh,
        scratch_types=[pltpu.VMEM((NCHK, CH), jnp.int32),
                       pltpu.VMEM((CH, H), jnp.float32),
                       pltpu.VMEM((CH,), jnp.float32),
                       pltpu.VMEM((CH,), jnp.float32),
                       pltpu.VMEM((CH,), jnp.int32),
                       pltpu.VMEM((CH,), jnp.int32),
                       pltpu.VMEM_SHARED((N, H), jnp.float32),
                       pltpu.VMEM_SHARED((DENP,), jnp.float32)],
    )
    def k(dst_h, rn_h, e0_h, e1_h, zn_h, zd_h, s_o, den_o,
          dst_v, r_b, e0_b, e1_b, i0_b, i1_b, acc, dsh):
        cid = lax.axis_index("c")
        sid = lax.axis_index("s")
        wid = cid * 16 + sid
        base = wid * EPW

        @pl.when(sid == 0)
        def _zero():
            pltpu.sync_copy(zn_h, acc)
            pltpu.sync_copy(zd_h, dsh)

        plsc.subcore_barrier()
        pltpu.sync_copy(dst_h.at[wid], dst_v)

        def body(i, carry):
            off = base + i * CH
            pltpu.sync_copy(rn_h.at[pl.ds(off, CH)], r_b)
            pltpu.sync_copy(e0_h.at[pl.ds(off, CH)], e0_b)
            pltpu.sync_copy(e1_h.at[pl.ds(off, CH)], e1_b)
            for g in range(CH // 16):
                d16 = dst_v[i, pl.ds(g * 16, 16)]
                i0_b[pl.ds(g * 16, 16)] = d16 * 2
                i1_b[pl.ds(g * 16, 16)] = d16 * 2 + 1
            pltpu.sync_copy(r_b, acc.at[dst_v.at[i]], add=True)
            pltpu.sync_copy(e0_b, dsh.at[i0_b], add=True)
            pltpu.sync_copy(e1_b, dsh.at[i1_b], add=True)
            return carry

        lax.fori_loop(0, NCHK, body, 0)
        plsc.subcore_barrier()

        @pl.when(sid == 0)
        def _out():
            pltpu.sync_copy(acc, s_o.at[cid])
            pltpu.sync_copy(dsh, den_o.at[pl.ds(cid * DENP, DENP)])

    return k(dst3, rnum, e0f, e1f, zn, zd)


def _final_body(s_r, d2_r, hin_r, hhn_r, scg_r, wo_r, bo_r, ctew_r, cteb_r,
                gw1_r, gb1_r, gw2_r, gb2_r, out_o):
    t = s_r[0, :, :] + s_r[1, :, :]
    d0 = d2_r[:, 0:1]
    d1 = d2_r[:, 1:2]
    a0 = t[:, 0:DH] / (d0 + 1e-16)
    a1 = t[:, DH:H] / (d1 + 1e-16)
    hin = hin_r[...]
    g = jax.nn.relu(jnp.dot(jnp.concatenate([a0, a1, hin], axis=1), wo_r[...],
                            precision=_HI) + bo_r[0:1, :])
    node_ts = scg_r[:, 0:1]
    hist_ts_next = scg_r[:, 1:2]
    rmask = scg_r[:, 2:3]
    dtg = jnp.maximum(node_ts - hist_ts_next, 0.0)
    te2 = jnp.cos(dtg * ctew_r[0:1, :] + cteb_r[0:1, :])
    hcat = jnp.concatenate([hhn_r[...], te2], axis=1)
    hc2 = jnp.dot(jax.nn.relu(jnp.dot(hcat, gw1_r[...], precision=_HI) + gb1_r[0:1, :]),
                  gw2_r[...], precision=_HI) + gb2_r[0:1, :]
    out_o[...] = jnp.where(rmask > 0.5, hc2, g)


def _final_pass(s, d2, hin, hhn, scg, wo, bo, ctew, cteb, gw1, gb1, gw2, gb2):
    full = lambda arr: pl.BlockSpec(arr.shape, lambda i: (0, 0))
    blk = lambda w: pl.BlockSpec((BN, w), lambda i: (i, 0))
    return pl.pallas_call(
        _final_body,
        grid=(NBN,),
        in_specs=[pl.BlockSpec((2, BN, H), lambda i: (0, i, 0)),
                  blk(8), blk(H), blk(H), blk(8),
                  full(wo), full(bo), full(ctew), full(cteb),
                  full(gw1), full(gb1), full(gw2), full(gb2)],
        out_specs=blk(H),
        out_shape=jax.ShapeDtypeStruct((N, H), jnp.float32),
    )(s, d2, hin, hhn, scg, wo, bo, ctew, cteb, gw1, gb1, gw2, gb2)


def kernel(x, memory, mem_ts, mailbox, mail_ts, node_ts, edge_index, edge_feat,
           edge_ts, h_mem_hist, hist_ts, is_remote_mem, h_hist_next,
           hist_ts_next, is_remote_next, te_w, te_b, gru_w_ih, gru_w_hh,
           gru_b_ih, gru_b_hh, nfm_w, nfm_b, cte_w, cte_b, mc_w1, mc_b1,
           mc_w2, mc_b2, gc_w1, gc_b1, gc_w2, gc_b2, Wq, Wk, Wv, Wo, bo):
    f32 = jnp.float32
    wiht = gru_w_ih.T
    whht = gru_w_hh.T
    bih = gru_b_ih.reshape(1, -1)
    bhh = gru_b_hh.reshape(1, -1)
    nfmt = nfm_w.T
    nfmb = nfm_b.reshape(1, -1)
    tew = te_w.reshape(1, -1)
    teb = te_b.reshape(1, -1)
    ctew = cte_w.reshape(1, -1)
    cteb = cte_b.reshape(1, -1)
    mb1 = mc_b1.reshape(1, -1)
    mb2 = mc_b2.reshape(1, -1)
    gb1 = gc_b1.reshape(1, -1)
    gb2 = gc_b2.reshape(1, -1)
    bo2 = bo.reshape(1, -1)
    wk_h8 = Wk[:H] * 0.125
    wk_et = Wk[H:]
    wv_h = Wv[:H]
    wvet = Wv[H:]
    wkv = jnp.concatenate([wk_h8, wv_h], axis=1)
    za = jnp.zeros((DH, 48), f32)
    wp8 = jnp.concatenate(
        [jnp.concatenate([wk_et[:, 0:DH].T * 0.125, za], axis=1),
         jnp.concatenate([za, wk_et[:, DH:H].T * 0.125], axis=1)], axis=0)
    wqp = jnp.concatenate(
        [Wq, jnp.dot(Wq, wp8, precision=_HI)], axis=1)

    sca = jnp.stack([mem_ts, mail_ts, node_ts, hist_ts,
                     is_remote_mem.astype(f32)], axis=1)
    sca = jnp.concatenate([sca, jnp.zeros((N, 3), f32)], axis=1)
    scg = jnp.stack([node_ts, hist_ts_next,
                     is_remote_next.astype(f32)], axis=1)
    scg = jnp.concatenate([scg, jnp.zeros((N, 5), f32)], axis=1)
    src3 = edge_index[0].astype(jnp.int32).reshape(NW, NCHK, CH)
    dst3 = edge_index[1].astype(jnp.int32).reshape(NW, NCHK, CH)
    ets = edge_ts.reshape(E, 1)
    zn = jnp.zeros((N, H), f32)
    zd = jnp.zeros((DENP,), f32)

    hin, gsv, dg = _node_pass(x, memory, mailbox, h_mem_hist, sca, wiht, whht,
                              bih, bhh, nfmt, nfmb, tew, teb, ctew, cteb,
                              mc_w1, mb1, mc_w2, mb2, wkv, wqp)
    kvs, dge = _sc_gather(gsv, dg, src3, dst3)
    rnum, epair = _edge_pass(kvs, dge, edge_feat, ets, tew, teb, wvet)
    e0f = epair[:, 0]
    e1f = epair[:, 1]
    s, den = _sc_scatter(dst3, rnum, e0f, e1f, zn, zd)
    dsum = (den[:DENP] + den[DENP:])[:DEN].reshape(N, 2)
    d2 = jnp.concatenate([dsum, jnp.zeros((N, 6), f32)], axis=1)
    return _final_pass(s, d2, hin, h_hist_next, scg, Wo, bo2, ctew, cteb,
                       gc_w1, gb1, gc_w2, gb2)


# speedup vs baseline: 17.5683x; 1.0138x over previous
import functools

import jax
import jax.numpy as jnp
from jax import lax
from jax.experimental import pallas as pl
from jax.experimental.pallas import tpu as pltpu
from jax.experimental.pallas import tpu_sc as plsc

N = 10000
E = 320000
H = 128
TF = 32
DE = 16
NH = 2
DH = 64

BN = 1000
NBN = N // BN
BE = 1600
NBE = E // BE
NW = 32
EPW = E // NW
CH = 80
NCHK = EPW // CH

DGW = 256
DEN = 2 * N
DENP = 20480

_HI = jax.lax.Precision.HIGHEST


def _node_body(x_r, mem_r, mbx_r, hmh_r, sca_r, wiht_r, whht_r, bih_r, bhh_r,
               nfmt_r, nfmb_r, tew_r, teb_r, ctew_r, cteb_r, mw1_r, mb1_r,
               mw2_r, mb2_r, wkv_r, wqp_r, hin_o, gsv_o, dg_o):
    mem_ts = sca_r[:, 0:1]
    mail_ts = sca_r[:, 1:2]
    node_ts = sca_r[:, 2:3]
    hist_ts = sca_r[:, 3:4]
    rmask = sca_r[:, 4:5]
    tf = jnp.cos((mail_ts - mem_ts) * tew_r[0:1, :] + teb_r[0:1, :])
    xin = jnp.concatenate([mbx_r[...], tf], axis=1)
    gi = jnp.dot(xin, wiht_r[...], precision=_HI) + bih_r[0:1, :]
    gh = jnp.dot(mem_r[...], whht_r[...], precision=_HI) + bhh_r[0:1, :]
    r = jax.nn.sigmoid(gi[:, 0:H] + gh[:, 0:H])
    z = jax.nn.sigmoid(gi[:, H:2 * H] + gh[:, H:2 * H])
    nn = jnp.tanh(gi[:, 2 * H:3 * H] + r * gh[:, 2 * H:3 * H])
    out_mem = (1.0 - z) * nn + z * mem_r[...]
    h = out_mem + jnp.dot(x_r[...], nfmt_r[...], precision=_HI) + nfmb_r[0:1, :]
    dtm = jnp.maximum(node_ts - hist_ts, 0.0)
    tec = jnp.cos(dtm * ctew_r[0:1, :] + cteb_r[0:1, :])
    hcat = jnp.concatenate([hmh_r[...], tec], axis=1)
    hc = jnp.dot(jax.nn.relu(jnp.dot(hcat, mw1_r[...], precision=_HI) + mb1_r[0:1, :]),
                 mw2_r[...], precision=_HI) + mb2_r[0:1, :]
    hin = jnp.where(rmask > 0.5, hc, h)
    hin_o[...] = hin
    gsv_o[...] = jnp.dot(hin, wkv_r[...], precision=_HI)
    qp = jnp.dot(hin, wqp_r[...], precision=_HI)
    dg_o[...] = jnp.concatenate(
        [qp, node_ts, jnp.zeros((qp.shape[0], DGW - 225), jnp.float32)], axis=1)


def _node_pass(x, memory, mailbox, h_mem_hist, sca, wiht, whht, bih, bhh,
               nfmt, nfmb, tew, teb, ctew, cteb, mw1, mb1, mw2, mb2, wkv, wqp):
    full = lambda arr: pl.BlockSpec(arr.shape, lambda i: (0, 0))
    blk = lambda w: pl.BlockSpec((BN, w), lambda i: (i, 0))
    return pl.pallas_call(
        _node_body,
        grid=(NBN,),
        in_specs=[blk(H), blk(H), blk(2 * H), blk(H), blk(8),
                  full(wiht), full(whht), full(bih), full(bhh),
                  full(nfmt), full(nfmb), full(tew), full(teb),
                  full(ctew), full(cteb), full(mw1), full(mb1),
                  full(mw2), full(mb2), full(wkv), full(wqp)],
        out_specs=[blk(H), blk(2 * H), blk(DGW)],
        out_shape=[jax.ShapeDtypeStruct((N, H), jnp.float32),
                   jax.ShapeDtypeStruct((N, 2 * H), jnp.float32),
                   jax.ShapeDtypeStruct((N, DGW), jnp.float32)],
    )(x, memory, mailbox, h_mem_hist, sca, wiht, whht, bih, bhh,
      nfmt, nfmb, tew, teb, ctew, cteb, mw1, mb1, mw2, mb2, wkv, wqp)


def _sc_gather(gsv, dg, src3, dst3):
    mesh = plsc.VectorSubcoreMesh(core_axis_name="c", subcore_axis_name="s")

    @functools.partial(
        pl.kernel,
        out_type=(jax.ShapeDtypeStruct((E, 2 * H), jnp.float32),
                  jax.ShapeDtypeStruct((E, DGW), jnp.float32)),
        mesh=mesh,
        scratch_types=[pltpu.VMEM((NCHK, CH), jnp.int32),
                       pltpu.VMEM((NCHK, CH), jnp.int32),
                       pltpu.VMEM((CH, 2 * H), jnp.float32),
                       pltpu.VMEM((CH, DGW), jnp.float32),
                       pltpu.SemaphoreType.DMA,
                       pltpu.SemaphoreType.DMA],
    )
    def k(gsv_h, dg_h, src_h, dst_h, kvs_o, dge_o, src_v, dst_v, kv_b, dg_b,
          sem1, sem2):
        wid = lax.axis_index("c") * 16 + lax.axis_index("s")
        base = wid * EPW
        pltpu.sync_copy(src_h.at[wid], src_v)
        pltpu.sync_copy(dst_h.at[wid], dst_v)

        def body(i, carry):
            off = base + i * CH
            c1 = pltpu.async_copy(gsv_h.at[src_v.at[i]], kv_b, sem1)
            c2 = pltpu.async_copy(dg_h.at[dst_v.at[i]], dg_b, sem2)
            c1.wait()
            c2.wait()
            pltpu.sync_copy(kv_b, kvs_o.at[pl.ds(off, CH)])
            pltpu.sync_copy(dg_b, dge_o.at[pl.ds(off, CH)])
            return carry

        lax.fori_loop(0, NCHK, body, 0)

    return k(gsv, dg, src3, dst3)


def _edge_body(kvs_r, dge_r, ef_r, ets_r, mcomb_r, tew_r, teb_r, wvet_r,
               rn_o, ep_o):
    dge = dge_r[...]
    kv = kvs_r[...]
    dt = dge[:, 224:225] - ets_r[...]
    te = jnp.cos(dt * tew_r[0:1, :] + teb_r[0:1, :])
    et = jnp.concatenate([ef_r[...], te], axis=1)
    et2 = jnp.concatenate([et, et], axis=1)
    a = dge[:, 0:H] * kv[:, 0:H]
    b = dge[:, H:224] * et2
    s2 = jnp.dot(jnp.concatenate([a, b], axis=1), mcomb_r[...], precision=_HI)
    ee = jnp.exp(jnp.minimum(s2, 60.0))
    e0 = ee[:, 0:1]
    e1 = ee[:, 1:2]
    v = kv[:, H:2 * H] + jnp.dot(et, wvet_r[...], precision=_HI)
    lane = jax.lax.broadcasted_iota(jnp.int32, (1, H), 1)
    ew = jnp.where(lane < DH, e0, e1)
    rn_o[...] = ew * v
    ep_o[...] = jnp.concatenate(
        [e0.reshape(BE // 8, 8), e1.reshape(BE // 8, 8)], axis=1)


def _edge_pass(kvs, dge, ef, ets, mcomb, tew, teb, wvet):
    full = lambda arr: pl.BlockSpec(arr.shape, lambda i: (0, 0))
    blk = lambda w: pl.BlockSpec((BE, w), lambda i: (i, 0))
    return pl.pallas_call(
        _edge_body,
        grid=(NBE,),
        in_specs=[blk(2 * H), blk(DGW), blk(DE), blk(1),
                  full(mcomb), full(tew), full(teb), full(wvet)],
        out_specs=[blk(H), pl.BlockSpec((BE // 8, 16), lambda i: (i, 0))],
        out_shape=[jax.ShapeDtypeStruct((E, H), jnp.float32),
                   jax.ShapeDtypeStruct((E // 8, 16), jnp.float32)],
    )(kvs, dge, ef, ets, mcomb, tew, teb, wvet)


def _sc_scatter(dst3, rnum, e0f, e1f, zn, zd):
    mesh = plsc.VectorSubcoreMesh(core_axis_name="c", subcore_axis_name="s")

    @functools.partial(
        pl.kernel,
        out_type=(jax.ShapeDtypeStruct((2, N, H), jnp.float32),
                  jax.ShapeDtypeStruct((2 * DENP,), jnp.float32)),
        mesh=mesh,
        scratch_types=[pltpu.VMEM((NCHK, CH), jnp.int32),
                       pltpu.VMEM((CH, H), jnp.float32),
                       pltpu.VMEM((CH,), jnp.float32),
                       pltpu.VMEM((CH,), jnp.float32),
                       pltpu.VMEM((CH,), jnp.int32),
                       pltpu.VMEM((CH,), jnp.int32),
                       pltpu.VMEM_SHARED((N, H), jnp.float32),
                       pltpu.VMEM_SHARED((DENP,), jnp.float32)],
    )
    def k(dst_h, rn_h, e0_h, e1_h, zn_h, zd_h, s_o, den_o,
          dst_v, r_b, e0_b, e1_b, i0_b, i1_b, acc, dsh):
        cid = lax.axis_index("c")
        sid = lax.axis_index("s")
        wid = cid * 16 + sid
        base = wid * EPW

        @pl.when(sid == 0)
        def _zero():
            pltpu.sync_copy(zn_h, acc)
            pltpu.sync_copy(zd_h, dsh)

        plsc.subcore_barrier()
        pltpu.sync_copy(dst_h.at[wid], dst_v)

        def body(i, carry):
            off = base + i * CH
            pltpu.sync_copy(rn_h.at[pl.ds(off, CH)], r_b)
            pltpu.sync_copy(e0_h.at[pl.ds(off, CH)], e0_b)
            pltpu.sync_copy(e1_h.at[pl.ds(off, CH)], e1_b)
            for g in range(CH // 16):
                d16 = dst_v[i, pl.ds(g * 16, 16)]
                i0_b[pl.ds(g * 16, 16)] = d16 * 2
                i1_b[pl.ds(g * 16, 16)] = d16 * 2 + 1
            pltpu.sync_copy(r_b, acc.at[dst_v.at[i]], add=True)
            pltpu.sync_copy(e0_b, dsh.at[i0_b], add=True)
            pltpu.sync_copy(e1_b, dsh.at[i1_b], add=True)
            return carry

        lax.fori_loop(0, NCHK, body, 0)
        plsc.subcore_barrier()

        @pl.when(sid == 0)
        def _out():
            pltpu.sync_copy(acc, s_o.at[cid])
            pltpu.sync_copy(dsh, den_o.at[pl.ds(cid * DENP, DENP)])

    return k(dst3, rnum, e0f, e1f, zn, zd)


def _final_body(s_r, d2_r, hin_r, hhn_r, scg_r, wo_r, bo_r, ctew_r, cteb_r,
                gw1_r, gb1_r, gw2_r, gb2_r, out_o):
    t = s_r[0, :, :] + s_r[1, :, :]
    d0 = d2_r[:, 0:1]
    d1 = d2_r[:, 1:2]
    lane = jax.lax.broadcasted_iota(jnp.int32, (1, H), 1)
    rcp = jnp.where(lane < DH, 1.0 / (d0 + 1e-16), 1.0 / (d1 + 1e-16))
    attn = t * rcp
    hin = hin_r[...]
    g = jax.nn.relu(jnp.dot(jnp.concatenate([attn, hin], axis=1), wo_r[...],
                            precision=_HI) + bo_r[0:1, :])
    node_ts = scg_r[:, 0:1]
    hist_ts_next = scg_r[:, 1:2]
    rmask = scg_r[:, 2:3]
    dtg = jnp.maximum(node_ts - hist_ts_next, 0.0)
    te2 = jnp.cos(dtg * ctew_r[0:1, :] + cteb_r[0:1, :])
    hcat = jnp.concatenate([hhn_r[...], te2], axis=1)
    hc2 = jnp.dot(jax.nn.relu(jnp.dot(hcat, gw1_r[...], precision=_HI) + gb1_r[0:1, :]),
                  gw2_r[...], precision=_HI) + gb2_r[0:1, :]
    out_o[...] = jnp.where(rmask > 0.5, hc2, g)


def _final_pass(s, d2, hin, hhn, scg, wo, bo, ctew, cteb, gw1, gb1, gw2, gb2):
    full = lambda arr: pl.BlockSpec(arr.shape, lambda i: (0, 0))
    blk = lambda w: pl.BlockSpec((BN, w), lambda i: (i, 0))
    return pl.pallas_call(
        _final_body,
        grid=(NBN,),
        in_specs=[pl.BlockSpec((2, BN, H), lambda i: (0, i, 0)),
                  blk(8), blk(H), blk(H), blk(8),
                  full(wo), full(bo), full(ctew), full(cteb),
                  full(gw1), full(gb1), full(gw2), full(gb2)],
        out_specs=blk(H),
        out_shape=jax.ShapeDtypeStruct((N, H), jnp.float32),
    )(s, d2, hin, hhn, scg, wo, bo, ctew, cteb, gw1, gb1, gw2, gb2)


def kernel(x, memory, mem_ts, mailbox, mail_ts, node_ts, edge_index, edge_feat,
           edge_ts, h_mem_hist, hist_ts, is_remote_mem, h_hist_next,
           hist_ts_next, is_remote_next, te_w, te_b, gru_w_ih, gru_w_hh,
           gru_b_ih, gru_b_hh, nfm_w, nfm_b, cte_w, cte_b, mc_w1, mc_b1,
           mc_w2, mc_b2, gc_w1, gc_b1, gc_w2, gc_b2, Wq, Wk, Wv, Wo, bo):
    f32 = jnp.float32
    wiht = gru_w_ih.T
    whht = gru_w_hh.T
    bih = gru_b_ih.reshape(1, -1)
    bhh = gru_b_hh.reshape(1, -1)
    nfmt = nfm_w.T
    nfmb = nfm_b.reshape(1, -1)
    tew = te_w.reshape(1, -1)
    teb = te_b.reshape(1, -1)
    ctew = cte_w.reshape(1, -1)
    cteb = cte_b.reshape(1, -1)
    mb1 = mc_b1.reshape(1, -1)
    mb2 = mc_b2.reshape(1, -1)
    gb1 = gc_b1.reshape(1, -1)
    gb2 = gc_b2.reshape(1, -1)
    bo2 = bo.reshape(1, -1)
    wk_h8 = Wk[:H] * 0.125
    wk_et = Wk[H:]
    wv_h = Wv[:H]
    wvet = Wv[H:]
    wkv = jnp.concatenate([wk_h8, wv_h], axis=1)
    za = jnp.zeros((DH, 48), f32)
    wp8 = jnp.concatenate(
        [jnp.concatenate([wk_et[:, 0:DH].T * 0.125, za], axis=1),
         jnp.concatenate([za, wk_et[:, DH:H].T * 0.125], axis=1)], axis=0)
    wqp = jnp.concatenate(
        [Wq, jnp.dot(Wq, wp8, precision=_HI)], axis=1)

    sca = jnp.stack([mem_ts, mail_ts, node_ts, hist_ts,
                     is_remote_mem.astype(f32)], axis=1)
    sca = jnp.concatenate([sca, jnp.zeros((N, 3), f32)], axis=1)
    scg = jnp.stack([node_ts, hist_ts_next,
                     is_remote_next.astype(f32)], axis=1)
    scg = jnp.concatenate([scg, jnp.zeros((N, 5), f32)], axis=1)
    src3 = edge_index[0].astype(jnp.int32).reshape(NW, NCHK, CH)
    dst3 = edge_index[1].astype(jnp.int32).reshape(NW, NCHK, CH)
    ets = edge_ts.reshape(E, 1)
    zn = jnp.zeros((N, H), f32)
    zd = jnp.zeros((DENP,), f32)
    mcomb = jnp.concatenate([
        jnp.repeat(jnp.eye(2, dtype=f32), DH, axis=0),
        jnp.repeat(jnp.eye(2, dtype=f32), 48, axis=0),
    ], axis=0)

    hin, gsv, dg = _node_pass(x, memory, mailbox, h_mem_hist, sca, wiht, whht,
                              bih, bhh, nfmt, nfmb, tew, teb, ctew, cteb,
                              mc_w1, mb1, mc_w2, mb2, wkv, wqp)
    kvs, dge = _sc_gather(gsv, dg, src3, dst3)
    rnum, ep = _edge_pass(kvs, dge, edge_feat, ets, mcomb, tew, teb, wvet)
    e0f = ep[:, 0:8].reshape(E)
    e1f = ep[:, 8:16].reshape(E)
    s, den = _sc_scatter(dst3, rnum, e0f, e1f, zn, zd)
    dsum = (den[:DENP] + den[DENP:])[:DEN].reshape(N, 2)
    d2 = jnp.concatenate([dsum, jnp.zeros((N, 6), f32)], axis=1)
    return _final_pass(s, d2, hin, h_hist_next, scg, Wo, bo2, ctew, cteb,
                       gc_w1, gb1, gc_w2, gb2)
